# Initial kernel scaffold; baseline (speedup 1.0000x reference)
import jax
import jax.numpy as jnp
from jax.experimental import pallas as pl


def kernel(x, node_type, edge_index, e_feat, W_fc, b_fc, W0, a0, ee0, W1, a1, ee1, Wres1, W2, a2, ee2, Wres2):
    raise NotImplementedError("write your pallas kernel here")


# speedup vs baseline: 43.5258x; 43.5258x over previous
import functools

import jax
import jax.numpy as jnp
from jax import lax
from jax.experimental import pallas as pl
from jax.experimental.pallas import tpu as pltpu
from jax.experimental.pallas import tpu_sc as plsc

N = 10000
E = 160000
D_IN = 128
NT = 3
NE = 5
H = 64
NC = 8
NEG = 0.05

NPAD = 10240
EPAD = 163840
CH = 128
RS = EPAD // 16 // CH
NCORE = 2
NSUB = 16
NSLICE = NPAD // NSUB

f32 = jnp.float32
i32 = jnp.int32


def _sc_edge_phase_a(core, sub, src_v, dst_v, et_v, ex_v, el_t, er_t, c_t,
                     ee_t, esum_sh, jidx_v=None, nt_t=None):
    def body(j, _):
        for sl in range(8):
            s = pl.ds(sl * 16, 16)
            sv = src_v[j, s]
            dv = dst_v[j, s]
            tv = et_v[j, s]
            e = (plsc.load_gather(el_t, [sv])
                 + plsc.load_gather(er_t, [dv])
                 + plsc.load_gather(ee_t, [tv]))
            e = jnp.where(e > 0, e, NEG * e)
            exv = jnp.exp(e - plsc.load_gather(c_t, [dv]))
            ex_v[j, s] = exv
            if jidx_v is not None:
                ntv = plsc.load_gather(nt_t, [sv])
                jidx_v[j, s] = dv * NT + ntv
        pltpu.sync_copy(ex_v.at[j], esum_sh.at[dst_v.at[j]], add=True)
        return 0

    lax.fori_loop(0, RS, body, 0)


def _sc_recip(sub, esum_sh, resum_sh, resum_t):
    sl = pl.ds(sub * NSLICE, NSLICE)
    pltpu.sync_copy(esum_sh.at[sl], resum_t.at[pl.ds(0, NSLICE)])
    for k in range(NSLICE // 16):
        s = pl.ds(k * 16, 16)
        resum_t[s] = 1.0 / (resum_t[s] + 1e-9)
    pltpu.sync_copy(resum_t.at[pl.ds(0, NSLICE)], resum_sh.at[sl])
    plsc.subcore_barrier()
    pltpu.sync_copy(resum_sh, resum_t)


def _sc_scale_rows(rows_v, ex_v, j, nq):
    iot = lax.broadcasted_iota(i32, (16,), 0)
    for g in range(8):
        a16 = ex_v[j, pl.ds(g * 16, 16)]
        for i in range(16):
            b = jnp.full((16,), jnp.sum(jnp.where(iot == i, a16, 0.0)), f32)
            for q in range(nq):
                s = pl.ds(q * 16, 16)
                rows_v[g * 16 + i, s] = rows_v[g * 16 + i, s] * b


def _sc_zero_shared(sub, rows_v, zbuf, esum_sh, acc_sh, acc_rows_per_sub, fw):
    for k in range(NSLICE // 16):
        zbuf[pl.ds(k * 16, 16)] = jnp.zeros((16,), f32)
    pltpu.sync_copy(zbuf, esum_sh.at[pl.ds(sub * NSLICE, NSLICE)])

    def zrow(i, _):
        for q in range(fw // 16):
            rows_v[i, pl.ds(q * 16, 16)] = jnp.zeros((16,), f32)
        return 0

    lax.fori_loop(0, CH, zrow, 0)
    for k in range(acc_rows_per_sub // CH):
        pltpu.sync_copy(
            rows_v, acc_sh.at[pl.ds(sub * acc_rows_per_sub + k * CH, CH)])


def _sc_attn_body(with_jidx, *refs):
    if with_jidx:
        (src_h, dst_h, et_h, nt_h, el_h, er_h, c_h, ee_h,
         alpha_h, jidx_h,
         src_v, dst_v, et_v, jidx_v, ex_v,
         el_t, er_t, c_t, nt_t, ee_t, resum_t,
         esum_sh, resum_sh) = refs
    else:
        (src_h, dst_h, et_h, el_h, er_h, c_h, ee_h,
         alpha_h,
         src_v, dst_v, et_v, ex_v,
         el_t, er_t, c_t, ee_t, resum_t,
         esum_sh, resum_sh) = refs
        nt_h = nt_t = jidx_v = jidx_h = None
    core = lax.axis_index("c")
    sub = lax.axis_index("s")
    base = sub * RS

    pltpu.sync_copy(el_h.at[core], el_t)
    pltpu.sync_copy(er_h.at[core], er_t)
    pltpu.sync_copy(c_h.at[core], c_t)
    pltpu.sync_copy(ee_h.at[core], ee_t)
    if with_jidx:
        pltpu.sync_copy(nt_h, nt_t)
    pltpu.sync_copy(src_h.at[pl.ds(base, RS)], src_v)
    pltpu.sync_copy(dst_h.at[pl.ds(base, RS)], dst_v)
    pltpu.sync_copy(et_h.at[pl.ds(base, RS)], et_v)

    for k in range(NSLICE // 16):
        resum_t[pl.ds(k * 16, 16)] = jnp.zeros((16,), f32)
    pltpu.sync_copy(resum_t.at[pl.ds(0, NSLICE)],
                    esum_sh.at[pl.ds(sub * NSLICE, NSLICE)])
    plsc.subcore_barrier()

    _sc_edge_phase_a(core, sub, src_v, dst_v, et_v, ex_v, el_t, er_t, c_t,
                     ee_t, esum_sh, jidx_v=jidx_v, nt_t=nt_t)
    plsc.subcore_barrier()

    _sc_recip(sub, esum_sh, resum_sh, resum_t)

    def pa2(j, _):
        for sl in range(8):
            s = pl.ds(sl * 16, 16)
            ex_v[j, s] = ex_v[j, s] * plsc.load_gather(resum_t, [dst_v[j, s]])
        return 0

    lax.fori_loop(0, RS, pa2, 0)
    pltpu.sync_copy(ex_v, alpha_h.at[core].at[pl.ds(base, RS)])
    if with_jidx:
        @pl.when(core == 0)
        def _():
            pltpu.sync_copy(jidx_v, jidx_h.at[pl.ds(base, RS)])


def _sc_attn(with_jidx, *args):
    outs = [jax.ShapeDtypeStruct((NCORE, EPAD // CH, CH), f32)]
    scr = [
        pltpu.VMEM((RS, CH), i32),
        pltpu.VMEM((RS, CH), i32),
        pltpu.VMEM((RS, CH), i32),
        pltpu.VMEM((RS, CH), f32),
        pltpu.VMEM((NPAD,), f32),
        pltpu.VMEM((NPAD,), f32),
        pltpu.VMEM((NPAD,), f32),
        pltpu.VMEM((8,), f32),
        pltpu.VMEM((NPAD,), f32),
        pltpu.VMEM_SHARED((NPAD,), f32),
        pltpu.VMEM_SHARED((NPAD,), f32),
    ]
    if with_jidx:
        outs.append(jax.ShapeDtypeStruct((EPAD // CH, CH), i32))
        scr.insert(3, pltpu.VMEM((RS, CH), i32))
        scr.insert(8, pltpu.VMEM((NPAD,), i32))
    kern = pl.kernel(
        functools.partial(_sc_attn_body, with_jidx),
        out_type=outs if with_jidx else outs[0],
        mesh=_sc_mesh(),
        compiler_params=pltpu.CompilerParams(needs_layout_passes=False,
                                             use_tc_tiling_on_sc=False),
        scratch_types=scr,
    )
    return kern(*args)


def _sc_msg_body(fw, nacc,
                 src_h, idx_h, alpha_h, tab_h, agg_h,
                 src_v, idx_v, al_v, rows_v,
                 acc_sh, sem):
    core = lax.axis_index("c")
    sub = lax.axis_index("s")
    base = sub * RS
    rps = nacc // NSUB

    pltpu.sync_copy(src_h.at[pl.ds(base, RS)], src_v)
    pltpu.sync_copy(idx_h.at[pl.ds(base, RS)], idx_v)
    pltpu.sync_copy(alpha_h.at[core].at[pl.ds(base, RS)], al_v)

    def zrow(i, _):
        for q in range(fw // 16):
            rows_v[i, pl.ds(q * 16, 16)] = jnp.zeros((16,), f32)
        return 0

    lax.fori_loop(0, CH, zrow, 0)
    for k in range(rps // CH):
        pltpu.sync_copy(rows_v, acc_sh.at[pl.ds(sub * rps + k * CH, CH)])
    plsc.subcore_barrier()

    def pb(j, _):
        pltpu.async_copy(tab_h.at[core].at[src_v.at[j]], rows_v, sem).wait()
        _sc_scale_rows(rows_v, al_v, j, fw // 16)
        pltpu.sync_copy(rows_v, acc_sh.at[idx_v.at[j]], add=True)
        return 0

    lax.fori_loop(0, RS, pb, 0)
    plsc.subcore_barrier()

    out_sl = pl.ds(sub * rps, rps)
    pltpu.sync_copy(acc_sh.at[out_sl], agg_h.at[core].at[out_sl])


def _sc_msg(fw, nacc, src2d, idx2d, alphaT, tab):
    kern = pl.kernel(
        functools.partial(_sc_msg_body, fw, nacc),
        out_type=jax.ShapeDtypeStruct((NCORE, nacc, fw), f32),
        mesh=_sc_mesh(),
        compiler_params=pltpu.CompilerParams(needs_layout_passes=False,
                                             use_tc_tiling_on_sc=False),
        scratch_types=[
            pltpu.VMEM((RS, CH), i32),
            pltpu.VMEM((RS, CH), i32),
            pltpu.VMEM((RS, CH), f32),
            pltpu.VMEM((CH, fw), f32),
            pltpu.VMEM_SHARED((nacc, fw), f32),
            pltpu.SemaphoreType.DMA,
        ],
    )
    return kern(src2d, idx2d, alphaT, tab)


def _sc_layer_post_body(fw, pb0, pbn,
                        src_h, dst_h, et_h, el_h, er_h, c_h, ee_h, feat_h,
                        agg_h,
                        src_v, dst_v, et_v, ex_v,
                        el_t, er_t, c_t, ee_t, resum_t, rows_v, zbuf,
                        esum_sh, resum_sh, acc_sh, sem):
    core = lax.axis_index("c")
    sub = lax.axis_index("s")
    base = sub * RS

    pltpu.sync_copy(el_h.at[core], el_t)
    pltpu.sync_copy(er_h.at[core], er_t)
    pltpu.sync_copy(c_h.at[core], c_t)
    pltpu.sync_copy(ee_h.at[core], ee_t)
    pltpu.sync_copy(src_h.at[pl.ds(base, RS)], src_v)
    pltpu.sync_copy(dst_h.at[pl.ds(base, RS)], dst_v)
    pltpu.sync_copy(et_h.at[pl.ds(base, RS)], et_v)

    _sc_zero_shared(sub, rows_v, zbuf, esum_sh, acc_sh, NSLICE, fw)
    plsc.subcore_barrier()

    _sc_edge_phase_a(core, sub, src_v, dst_v, et_v, ex_v, el_t, er_t, c_t,
                     ee_t, esum_sh)
    plsc.subcore_barrier()

    _sc_recip(sub, esum_sh, resum_sh, resum_t)

    jlo = core * pb0

    def pb(jj, _):
        j = jlo + jj
        for sl in range(8):
            s = pl.ds(sl * 16, 16)
            ex_v[j, s] = ex_v[j, s] * plsc.load_gather(resum_t, [dst_v[j, s]])
        pltpu.async_copy(feat_h.at[core].at[src_v.at[j]], rows_v, sem).wait()
        _sc_scale_rows(rows_v, ex_v, j, fw // 16)
        pltpu.sync_copy(rows_v, acc_sh.at[dst_v.at[j]], add=True)
        return 0

    lax.fori_loop(0, pbn, pb, 0)
    plsc.subcore_barrier()

    out_sl = pl.ds(sub * NSLICE, NSLICE)
    pltpu.sync_copy(acc_sh.at[out_sl], agg_h.at[core].at[out_sl])


@functools.cache
def _sc_mesh():
    return plsc.VectorSubcoreMesh(core_axis_name="c", subcore_axis_name="s",
                                  num_cores=NCORE, num_subcores=NSUB)


def _sc_layer_post(fw, pb0, pbn, src2d, dst2d, et2d, elT, erT, cT, eeT, featT):
    body = functools.partial(_sc_layer_post_body, fw, pb0, pbn)
    kern = pl.kernel(
        body,
        out_type=jax.ShapeDtypeStruct((NCORE, NPAD, fw), f32),
        mesh=_sc_mesh(),
        compiler_params=pltpu.CompilerParams(needs_layout_passes=False,
                                             use_tc_tiling_on_sc=False),
        scratch_types=[
            pltpu.VMEM((RS, CH), i32),
            pltpu.VMEM((RS, CH), i32),
            pltpu.VMEM((RS, CH), i32),
            pltpu.VMEM((RS, CH), f32),
            pltpu.VMEM((NPAD,), f32),
            pltpu.VMEM((NPAD,), f32),
            pltpu.VMEM((NPAD,), f32),
            pltpu.VMEM((8,), f32),
            pltpu.VMEM((NPAD,), f32),
            pltpu.VMEM((CH, fw), f32),
            pltpu.VMEM((NSLICE,), f32),
            pltpu.VMEM_SHARED((NPAD,), f32),
            pltpu.VMEM_SHARED((NPAD,), f32),
            pltpu.VMEM_SHARED((NPAD, fw), f32),
            pltpu.SemaphoreType.DMA,
        ],
    )
    return kern(src2d, dst2d, et2d, elT, erT, cT, eeT, featT)


BN = 1024
GRID = NPAD // BN


def _elu(v):
    return jnp.where(v > 0, v, jnp.exp(v) - 1.0)


def _tc1_body(x_ref, nt_ref, wfc_ref, bfc_ref, w0_ref, a0_ref,
              hca_ref, hcb_ref, elw_ref):
    xb = x_ref[...]
    ntb = nt_ref[...]
    hc = jnp.zeros((BN, H), f32)
    els = []
    ers = []
    for hd in range(2):
        els.append(jnp.zeros((BN, 1), f32))
        ers.append(jnp.zeros((BN, 1), f32))
    for t in range(NT):
        pt = jnp.dot(xb, wfc_ref[t], preferred_element_type=f32) + bfc_ref[t][None, :]
        mt = (ntb == t).astype(f32)
        ht = pt * mt
        hc = hc + ht
        ft = jnp.dot(ht, w0_ref[...], preferred_element_type=f32)
        for hd in range(2):
            fslice = ft[:, hd * H:(hd + 1) * H]
            els[hd] = els[hd] + jnp.dot(fslice, a0_ref[0, hd, t][:, None],
                                        preferred_element_type=f32)
            ers[hd] = ers[hd] + jnp.dot(fslice, a0_ref[1, hd, t][:, None],
                                        preferred_element_type=f32)
    hca_ref[...] = jnp.stack([hc[:, :H // 2], hc[:, :H // 2]], axis=0)
    hcb_ref[...] = jnp.stack([hc[:, H // 2:], hc[:, H // 2:]], axis=0)
    elw_ref[...] = jnp.concatenate(
        els + ers + [jnp.zeros((BN, 4), f32)], axis=1)


def _tc2_body(agga_ref, aggb_ref, w0_ref, w1_ref, a1_ref,
              h1_ref, feata_ref, featb_ref, elw_ref):
    h1_parts = []
    for hd in range(2):
        a = jnp.concatenate([agga_ref[hd], aggb_ref[hd]], axis=1)
        v = jnp.dot(a, w0_ref[...][:, hd * H:(hd + 1) * H],
                    preferred_element_type=f32)
        h1_parts.append(_elu(v).reshape(BN, NT, H))
    h1 = jnp.concatenate(h1_parts, axis=2)
    h1_ref[...] = h1
    els = [jnp.zeros((BN, 1), f32) for _ in range(2)]
    ers = [jnp.zeros((BN, 1), f32) for _ in range(2)]
    fheads = [[], []]
    for t in range(NT):
        ft = jnp.dot(h1[:, t, :], w1_ref[...], preferred_element_type=f32)
        for hd in range(2):
            fslice = ft[:, hd * H:(hd + 1) * H]
            fheads[hd].append(fslice)
            els[hd] = els[hd] + jnp.dot(fslice, a1_ref[0, hd, t][:, None],
                                        preferred_element_type=f32)
            ers[hd] = ers[hd] + jnp.dot(fslice, a1_ref[1, hd, t][:, None],
                                        preferred_element_type=f32)
    f0 = jnp.concatenate(fheads[0], axis=1)
    f1 = jnp.concatenate(fheads[1], axis=1)
    hw = NT * H // 2
    feata_ref[...] = jnp.stack([f0[:, :hw], f1[:, :hw]], axis=0)
    featb_ref[...] = jnp.stack([f0[:, hw:], f1[:, hw:]], axis=0)
    elw_ref[...] = jnp.concatenate(
        els + ers + [jnp.zeros((BN, 4), f32)], axis=1)


def _tc3_body(agga_ref, aggb_ref, h1_ref, wres1_ref, w2_ref, a2_ref,
              h2_ref, feat_ref, elw_ref):
    h1 = h1_ref[...]
    h2_parts = []
    for hd in range(2):
        a = jnp.concatenate([agga_ref[hd], aggb_ref[hd]], axis=1)
        cols = []
        for t in range(NT):
            res = jnp.dot(h1[:, t, :],
                          wres1_ref[...][:, hd * H:(hd + 1) * H],
                          preferred_element_type=f32)
            cols.append(_elu(a[:, t * H:(t + 1) * H] + res)[:, None, :])
        h2_parts.append(jnp.concatenate(cols, axis=1))
    h2 = jnp.concatenate(h2_parts, axis=2)
    h2_ref[...] = h2
    el = jnp.zeros((BN, 1), f32)
    er = jnp.zeros((BN, 1), f32)
    fcols = []
    for t in range(NT):
        ft = jnp.dot(h2[:, t, :], w2_ref[...], preferred_element_type=f32)
        fcols.append(ft)
        el = el + jnp.dot(ft, a2_ref[0, 0, t][:, None],
                          preferred_element_type=f32)
        er = er + jnp.dot(ft, a2_ref[1, 0, t][:, None],
                          preferred_element_type=f32)
    fcat = jnp.concatenate(fcols + [jnp.zeros((BN, 32 - NT * NC), f32)],
                           axis=1)
    feat_ref[...] = jnp.stack([fcat, fcat], axis=0)
    elw_ref[...] = jnp.concatenate(
        [el, el, er, er, jnp.zeros((BN, 4), f32)], axis=1)


def _tc4_body(agg_ref, h2_ref, wres2_ref, out_ref):
    s = agg_ref[0][:, :NT * NC] + agg_ref[1][:, :NT * NC]
    h2 = h2_ref[...]
    acc = jnp.zeros((BN, NC), f32)
    for t in range(NT):
        res = jnp.dot(h2[:, t, :], wres2_ref[...], preferred_element_type=f32)
        acc = acc + s[:, t * NC:(t + 1) * NC] + res
    out_ref[...] = acc * (1.0 / NT)


def _full(shape):
    return pl.BlockSpec(shape, lambda i: tuple(0 for _ in shape))


def _tc1(xp, ntp2, W_fc, b_fc, W0, a0r):
    return pl.pallas_call(
        _tc1_body,
        grid=(GRID,),
        in_specs=[
            pl.BlockSpec((BN, D_IN), lambda i: (i, 0)),
            pl.BlockSpec((BN, 1), lambda i: (i, 0)),
            _full((NT, D_IN, H)),
            _full((NT, H)),
            _full((H, 2 * H)),
            _full((2, 2, NT, H)),
        ],
        out_specs=[
            pl.BlockSpec((2, BN, H // 2), lambda i: (0, i, 0)),
            pl.BlockSpec((2, BN, H // 2), lambda i: (0, i, 0)),
            pl.BlockSpec((BN, 8), lambda i: (i, 0)),
        ],
        out_shape=[
            jax.ShapeDtypeStruct((2, NPAD, H // 2), f32),
            jax.ShapeDtypeStruct((2, NPAD, H // 2), f32),
            jax.ShapeDtypeStruct((NPAD, 8), f32),
        ],
    )(xp, ntp2, W_fc, b_fc, W0, a0r)


def _tc2(agg0a, agg0b, W0, W1, a1r):
    hw = NT * H // 2
    return pl.pallas_call(
        _tc2_body,
        grid=(GRID,),
        in_specs=[
            pl.BlockSpec((2, NT * BN, H // 2), lambda i: (0, i, 0)),
            pl.BlockSpec((2, NT * BN, H // 2), lambda i: (0, i, 0)),
            _full((H, 2 * H)),
            _full((2 * H, 2 * H)),
            _full((2, 2, NT, H)),
        ],
        out_specs=[
            pl.BlockSpec((BN, NT, 2 * H), lambda i: (i, 0, 0)),
            pl.BlockSpec((2, BN, hw), lambda i: (0, i, 0)),
            pl.BlockSpec((2, BN, hw), lambda i: (0, i, 0)),
            pl.BlockSpec((BN, 8), lambda i: (i, 0)),
        ],
        out_shape=[
            jax.ShapeDtypeStruct((NPAD, NT, 2 * H), f32),
            jax.ShapeDtypeStruct((2, NPAD, hw), f32),
            jax.ShapeDtypeStruct((2, NPAD, hw), f32),
            jax.ShapeDtypeStruct((NPAD, 8), f32),
        ],
    )(agg0a, agg0b, W0, W1, a1r)


def _tc3(agg1a, agg1b, h1, Wres1, W2, a2r):
    hw = NT * H // 2
    return pl.pallas_call(
        _tc3_body,
        grid=(GRID,),
        in_specs=[
            pl.BlockSpec((2, BN, hw), lambda i: (0, i, 0)),
            pl.BlockSpec((2, BN, hw), lambda i: (0, i, 0)),
            pl.BlockSpec((BN, NT, 2 * H), lambda i: (i, 0, 0)),
            _full((2 * H, 2 * H)),
            _full((2 * H, NC)),
            _full((2, 1, NT, NC)),
        ],
        out_specs=[
            pl.BlockSpec((BN, NT, 2 * H), lambda i: (i, 0, 0)),
            pl.BlockSpec((2, BN, 32), lambda i: (0, i, 0)),
            pl.BlockSpec((BN, 8), lambda i: (i, 0)),
        ],
        out_shape=[
            jax.ShapeDtypeStruct((NPAD, NT, 2 * H), f32),
            jax.ShapeDtypeStruct((2, NPAD, 32), f32),
            jax.ShapeDtypeStruct((NPAD, 8), f32),
        ],
    )(agg1a, agg1b, h1, Wres1, W2, a2r)


def _tc4(agg2, h2, Wres2):
    return pl.pallas_call(
        _tc4_body,
        grid=(GRID,),
        in_specs=[
            pl.BlockSpec((2, BN, 32), lambda i: (0, i, 0)),
            pl.BlockSpec((BN, NT, 2 * H), lambda i: (i, 0, 0)),
            _full((2 * H, NC)),
        ],
        out_specs=pl.BlockSpec((BN, NC), lambda i: (i, 0)),
        out_shape=jax.ShapeDtypeStruct((NPAD, NC), f32),
    )(agg2, h2, Wres2)


def _stab(elw, eeT, nheads):
    elT = elw[:, 0:2].T
    erT = elw[:, 2:4].T
    m = (jnp.max(elT, axis=1, keepdims=True)
         + jnp.max(eeT[:, :NE], axis=1, keepdims=True))
    z = m + erT
    cT = jnp.where(z > 0, z, NEG * z)
    return elT, erT, cT


def kernel(x, node_type, edge_index, e_feat, W_fc, b_fc, W0, a0, ee0,
           W1, a1, ee1, Wres1, W2, a2, ee2, Wres2):
    src = edge_index[0]
    dst = edge_index[1]

    pad = EPAD - E
    src2d = jnp.concatenate([src, jnp.zeros((pad,), i32)]).reshape(-1, CH)
    dst2d = jnp.concatenate([dst, jnp.zeros((pad,), i32)]).reshape(-1, CH)
    et2d = jnp.concatenate(
        [e_feat, jnp.full((pad,), NE, i32)]).reshape(-1, CH)

    xp = jnp.concatenate([x, jnp.zeros((NPAD - N, D_IN), f32)], axis=0)
    ntp = jnp.concatenate([node_type, jnp.zeros((NPAD - N,), i32)])
    ntp2 = ntp[:, None]

    def ee_table(ee, heads):
        cols = [ee[:, hd if heads == 2 else 0][None, :] for hd in range(2)]
        t = jnp.concatenate(cols, axis=0)
        return jnp.concatenate([t, jnp.full((2, 8 - NE), -1e30, f32)], axis=1)

    a0r = a0.reshape(2, 2, NT, H)
    a1r = a1.reshape(2, 2, NT, H)
    a2r = a2.reshape(2, 1, NT, NC)

    hca, hcb, elw0 = _tc1(xp, ntp2, W_fc, b_fc, W0, a0r)
    ee0T = ee_table(ee0, 2)
    elT0, erT0, cT0 = _stab(elw0, ee0T, 2)
    alpha0, jidx0 = _sc_attn(True, src2d, dst2d, et2d, ntp,
                             elT0, erT0, cT0, ee0T)
    agg0a = _sc_msg(H // 2, NT * NPAD, src2d, jidx0, alpha0, hca)
    agg0b = _sc_msg(H // 2, NT * NPAD, src2d, jidx0, alpha0, hcb)

    h1, feat1a, feat1b, elw1 = _tc2(agg0a, agg0b, W0, W1, a1r)
    ee1T = ee_table(ee1, 2)
    elT1, erT1, cT1 = _stab(elw1, ee1T, 2)
    alpha1 = _sc_attn(False, src2d, dst2d, et2d, elT1, erT1, cT1, ee1T)
    agg1a = _sc_msg(NT * H // 2, NPAD, src2d, dst2d, alpha1, feat1a)
    agg1b = _sc_msg(NT * H // 2, NPAD, src2d, dst2d, alpha1, feat1b)

    h2, feat2, elw2 = _tc3(agg1a, agg1b, h1, Wres1, W2, a2r)
    ee2T = ee_table(ee2, 1)
    elT2, erT2, cT2 = _stab(elw2, ee2T, 1)
    agg2 = _sc_layer_post(32, RS // 2, RS // 2, src2d, dst2d, et2d,
                          elT2, erT2, cT2, ee2T, feat2)

    out = _tc4(agg2, h2, Wres2)
    return out[:N]


# speedup vs baseline: 45.0441x; 1.0349x over previous
import functools

import jax
import jax.numpy as jnp
from jax import lax
from jax.experimental import pallas as pl
from jax.experimental.pallas import tpu as pltpu
from jax.experimental.pallas import tpu_sc as plsc

N = 10000
E = 160000
D_IN = 128
NT = 3
NE = 5
H = 64
NC = 8
NEG = 0.05

NPAD = 10240
EPAD = 163840
CH = 128
RS = EPAD // 16 // CH
NCORE = 2
NSUB = 16
NSLICE = NPAD // NSUB

f32 = jnp.float32
i32 = jnp.int32


def _sc_edge_phase_a(core, sub, src_v, dst_v, et_v, ex_v, el_t, er_t, c_t,
                     ee_t, esum_sh, jidx_v=None, nt_t=None):
    def body(j, _):
        for sl in range(8):
            s = pl.ds(sl * 16, 16)
            sv = src_v[j, s]
            dv = dst_v[j, s]
            tv = et_v[j, s]
            e = (plsc.load_gather(el_t, [sv])
                 + plsc.load_gather(er_t, [dv])
                 + plsc.load_gather(ee_t, [tv]))
            e = jnp.where(e > 0, e, NEG * e)
            exv = jnp.exp(e - plsc.load_gather(c_t, [dv]))
            ex_v[j, s] = exv
            if jidx_v is not None:
                ntv = plsc.load_gather(nt_t, [sv])
                jidx_v[j, s] = dv * NT + ntv
        pltpu.sync_copy(ex_v.at[j], esum_sh.at[dst_v.at[j]], add=True)
        return 0

    lax.fori_loop(0, RS, body, 0)


def _sc_recip(sub, esum_sh, resum_sh, resum_t):
    sl = pl.ds(sub * NSLICE, NSLICE)
    pltpu.sync_copy(esum_sh.at[sl], resum_t.at[pl.ds(0, NSLICE)])
    for k in range(NSLICE // 16):
        s = pl.ds(k * 16, 16)
        resum_t[s] = 1.0 / (resum_t[s] + 1e-9)
    pltpu.sync_copy(resum_t.at[pl.ds(0, NSLICE)], resum_sh.at[sl])
    plsc.subcore_barrier()
    pltpu.sync_copy(resum_sh, resum_t)


def _sc_scale_rows(rows_v, ex_v, j, nq):
    jv = jnp.full((16,), j, i32)
    for i in range(CH):
        b = plsc.load_gather(ex_v, [jv, jnp.full((16,), i, i32)])
        for q in range(nq):
            s = pl.ds(q * 16, 16)
            rows_v[i, s] = rows_v[i, s] * b


def _sc_zero_shared(sub, rows_v, zbuf, esum_sh, acc_sh, acc_rows_per_sub, fw):
    for k in range(NSLICE // 16):
        zbuf[pl.ds(k * 16, 16)] = jnp.zeros((16,), f32)
    pltpu.sync_copy(zbuf, esum_sh.at[pl.ds(sub * NSLICE, NSLICE)])

    def zrow(i, _):
        for q in range(fw // 16):
            rows_v[i, pl.ds(q * 16, 16)] = jnp.zeros((16,), f32)
        return 0

    lax.fori_loop(0, CH, zrow, 0)
    for k in range(acc_rows_per_sub // CH):
        pltpu.sync_copy(
            rows_v, acc_sh.at[pl.ds(sub * acc_rows_per_sub + k * CH, CH)])


def _sc_attn_body(with_jidx, *refs):
    if with_jidx:
        (src_h, dst_h, et_h, nt_h, el_h, er_h, c_h, ee_h,
         alpha_h, jidx_h,
         src_v, dst_v, et_v, jidx_v, ex_v,
         el_t, er_t, c_t, nt_t, ee_t, resum_t,
         esum_sh, resum_sh) = refs
    else:
        (src_h, dst_h, et_h, el_h, er_h, c_h, ee_h,
         alpha_h,
         src_v, dst_v, et_v, ex_v,
         el_t, er_t, c_t, ee_t, resum_t,
         esum_sh, resum_sh) = refs
        nt_h = nt_t = jidx_v = jidx_h = None
    core = lax.axis_index("c")
    sub = lax.axis_index("s")
    base = sub * RS

    pltpu.sync_copy(el_h.at[core], el_t)
    pltpu.sync_copy(er_h.at[core], er_t)
    pltpu.sync_copy(c_h.at[core], c_t)
    pltpu.sync_copy(ee_h.at[core], ee_t)
    if with_jidx:
        pltpu.sync_copy(nt_h, nt_t)
    pltpu.sync_copy(src_h.at[pl.ds(base, RS)], src_v)
    pltpu.sync_copy(dst_h.at[pl.ds(base, RS)], dst_v)
    pltpu.sync_copy(et_h.at[pl.ds(base, RS)], et_v)

    for k in range(NSLICE // 16):
        resum_t[pl.ds(k * 16, 16)] = jnp.zeros((16,), f32)
    pltpu.sync_copy(resum_t.at[pl.ds(0, NSLICE)],
                    esum_sh.at[pl.ds(sub * NSLICE, NSLICE)])
    plsc.subcore_barrier()

    _sc_edge_phase_a(core, sub, src_v, dst_v, et_v, ex_v, el_t, er_t, c_t,
                     ee_t, esum_sh, jidx_v=jidx_v, nt_t=nt_t)
    plsc.subcore_barrier()

    _sc_recip(sub, esum_sh, resum_sh, resum_t)

    def pa2(j, _):
        for sl in range(8):
            s = pl.ds(sl * 16, 16)
            ex_v[j, s] = ex_v[j, s] * plsc.load_gather(resum_t, [dst_v[j, s]])
        return 0

    lax.fori_loop(0, RS, pa2, 0)
    pltpu.sync_copy(ex_v, alpha_h.at[core].at[pl.ds(base, RS)])
    if with_jidx:
        @pl.when(core == 0)
        def _():
            pltpu.sync_copy(jidx_v, jidx_h.at[pl.ds(base, RS)])


def _sc_attn(with_jidx, *args):
    outs = [jax.ShapeDtypeStruct((NCORE, EPAD // CH, CH), f32)]
    scr = [
        pltpu.VMEM((RS, CH), i32),
        pltpu.VMEM((RS, CH), i32),
        pltpu.VMEM((RS, CH), i32),
        pltpu.VMEM((RS, CH), f32),
        pltpu.VMEM((NPAD,), f32),
        pltpu.VMEM((NPAD,), f32),
        pltpu.VMEM((NPAD,), f32),
        pltpu.VMEM((8,), f32),
        pltpu.VMEM((NPAD,), f32),
        pltpu.VMEM_SHARED((NPAD,), f32),
        pltpu.VMEM_SHARED((NPAD,), f32),
    ]
    if with_jidx:
        outs.append(jax.ShapeDtypeStruct((EPAD // CH, CH), i32))
        scr.insert(3, pltpu.VMEM((RS, CH), i32))
        scr.insert(8, pltpu.VMEM((NPAD,), i32))
    kern = pl.kernel(
        functools.partial(_sc_attn_body, with_jidx),
        out_type=outs if with_jidx else outs[0],
        mesh=_sc_mesh(),
        compiler_params=pltpu.CompilerParams(needs_layout_passes=False,
                                             use_tc_tiling_on_sc=False),
        scratch_types=scr,
    )
    return kern(*args)


def _sc_msg_body(fw, nacc,
                 src_h, idx_h, alpha_h, tab_h, agg_h,
                 src_v, idx_v, al_v, rows0, rows1,
                 acc_sh, sem0, sem1):
    core = lax.axis_index("c")
    sub = lax.axis_index("s")
    base = sub * RS
    rps = nacc // NSUB
    rows = (rows0, rows1)
    sems = (sem0, sem1)
    tab = tab_h.at[core]

    pltpu.sync_copy(src_h.at[pl.ds(base, RS)], src_v)
    pltpu.sync_copy(idx_h.at[pl.ds(base, RS)], idx_v)
    pltpu.sync_copy(alpha_h.at[core].at[pl.ds(base, RS)], al_v)

    def zrow(i, _):
        for q in range(fw // 16):
            rows0[i, pl.ds(q * 16, 16)] = jnp.zeros((16,), f32)
        return 0

    lax.fori_loop(0, CH, zrow, 0)
    for k in range(rps // CH):
        pltpu.sync_copy(rows0, acc_sh.at[pl.ds(sub * rps + k * CH, CH)])
    plsc.subcore_barrier()

    pltpu.async_copy(tab.at[src_v.at[0]], rows0, sem0)

    def pb2(j2, _):
        for b in range(2):
            j = j2 * 2 + b
            nb = 1 - b
            pltpu.make_async_copy(tab.at[src_v.at[j]], rows[b], sems[b]).wait()
            if b == 0:
                pltpu.async_copy(tab.at[src_v.at[j + 1]], rows[nb], sems[nb])
            else:
                @pl.when(j2 < RS // 2 - 1)
                def _():
                    pltpu.async_copy(tab.at[src_v.at[j + 1]], rows[nb],
                                     sems[nb])
            _sc_scale_rows(rows[b], al_v, j, fw // 16)
            pltpu.sync_copy(rows[b], acc_sh.at[idx_v.at[j]], add=True)
        return 0

    lax.fori_loop(0, RS // 2, pb2, 0)
    plsc.subcore_barrier()

    out_sl = pl.ds(sub * rps, rps)
    pltpu.sync_copy(acc_sh.at[out_sl], agg_h.at[core].at[out_sl])


def _sc_msg(fw, nacc, src2d, idx2d, alphaT, tab):
    kern = pl.kernel(
        functools.partial(_sc_msg_body, fw, nacc),
        out_type=jax.ShapeDtypeStruct((NCORE, nacc, fw), f32),
        mesh=_sc_mesh(),
        compiler_params=pltpu.CompilerParams(needs_layout_passes=False,
                                             use_tc_tiling_on_sc=False),
        scratch_types=[
            pltpu.VMEM((RS, CH), i32),
            pltpu.VMEM((RS, CH), i32),
            pltpu.VMEM((RS, CH), f32),
            pltpu.VMEM((CH, fw), f32),
            pltpu.VMEM((CH, fw), f32),
            pltpu.VMEM_SHARED((nacc, fw), f32),
            pltpu.SemaphoreType.DMA,
            pltpu.SemaphoreType.DMA,
        ],
    )
    return kern(src2d, idx2d, alphaT, tab)


def _sc_layer_post_body(fw, pb0, pbn,
                        src_h, dst_h, et_h, el_h, er_h, c_h, ee_h, feat_h,
                        agg_h,
                        src_v, dst_v, et_v, ex_v,
                        el_t, er_t, c_t, ee_t, resum_t, rows_v, zbuf,
                        esum_sh, resum_sh, acc_sh, sem):
    core = lax.axis_index("c")
    sub = lax.axis_index("s")
    base = sub * RS

    pltpu.sync_copy(el_h.at[core], el_t)
    pltpu.sync_copy(er_h.at[core], er_t)
    pltpu.sync_copy(c_h.at[core], c_t)
    pltpu.sync_copy(ee_h.at[core], ee_t)
    pltpu.sync_copy(src_h.at[pl.ds(base, RS)], src_v)
    pltpu.sync_copy(dst_h.at[pl.ds(base, RS)], dst_v)
    pltpu.sync_copy(et_h.at[pl.ds(base, RS)], et_v)

    _sc_zero_shared(sub, rows_v, zbuf, esum_sh, acc_sh, NSLICE, fw)
    plsc.subcore_barrier()

    _sc_edge_phase_a(core, sub, src_v, dst_v, et_v, ex_v, el_t, er_t, c_t,
                     ee_t, esum_sh)
    plsc.subcore_barrier()

    _sc_recip(sub, esum_sh, resum_sh, resum_t)

    jlo = core * pb0

    def pb(jj, _):
        j = jlo + jj
        for sl in range(8):
            s = pl.ds(sl * 16, 16)
            ex_v[j, s] = ex_v[j, s] * plsc.load_gather(resum_t, [dst_v[j, s]])
        pltpu.async_copy(feat_h.at[core].at[src_v.at[j]], rows_v, sem).wait()
        _sc_scale_rows(rows_v, ex_v, j, fw // 16)
        pltpu.sync_copy(rows_v, acc_sh.at[dst_v.at[j]], add=True)
        return 0

    lax.fori_loop(0, pbn, pb, 0)
    plsc.subcore_barrier()

    out_sl = pl.ds(sub * NSLICE, NSLICE)
    pltpu.sync_copy(acc_sh.at[out_sl], agg_h.at[core].at[out_sl])


@functools.cache
def _sc_mesh():
    return plsc.VectorSubcoreMesh(core_axis_name="c", subcore_axis_name="s",
                                  num_cores=NCORE, num_subcores=NSUB)


def _sc_layer_post(fw, pb0, pbn, src2d, dst2d, et2d, elT, erT, cT, eeT, featT):
    body = functools.partial(_sc_layer_post_body, fw, pb0, pbn)
    kern = pl.kernel(
        body,
        out_type=jax.ShapeDtypeStruct((NCORE, NPAD, fw), f32),
        mesh=_sc_mesh(),
        compiler_params=pltpu.CompilerParams(needs_layout_passes=False,
                                             use_tc_tiling_on_sc=False),
        scratch_types=[
            pltpu.VMEM((RS, CH), i32),
            pltpu.VMEM((RS, CH), i32),
            pltpu.VMEM((RS, CH), i32),
            pltpu.VMEM((RS, CH), f32),
            pltpu.VMEM((NPAD,), f32),
            pltpu.VMEM((NPAD,), f32),
            pltpu.VMEM((NPAD,), f32),
            pltpu.VMEM((8,), f32),
            pltpu.VMEM((NPAD,), f32),
            pltpu.VMEM((CH, fw), f32),
            pltpu.VMEM((NSLICE,), f32),
            pltpu.VMEM_SHARED((NPAD,), f32),
            pltpu.VMEM_SHARED((NPAD,), f32),
            pltpu.VMEM_SHARED((NPAD, fw), f32),
            pltpu.SemaphoreType.DMA,
        ],
    )
    return kern(src2d, dst2d, et2d, elT, erT, cT, eeT, featT)


BN = 1024
GRID = NPAD // BN


def _elu(v):
    return jnp.where(v > 0, v, jnp.exp(v) - 1.0)


def _tc1_body(x_ref, nt_ref, wfc_ref, bfc_ref, w0_ref, a0_ref,
              hca_ref, hcb_ref, elw_ref):
    xb = x_ref[...]
    ntb = nt_ref[...]
    hc = jnp.zeros((BN, H), f32)
    els = []
    ers = []
    for hd in range(2):
        els.append(jnp.zeros((BN, 1), f32))
        ers.append(jnp.zeros((BN, 1), f32))
    for t in range(NT):
        pt = jnp.dot(xb, wfc_ref[t], preferred_element_type=f32) + bfc_ref[t][None, :]
        mt = (ntb == t).astype(f32)
        ht = pt * mt
        hc = hc + ht
        ft = jnp.dot(ht, w0_ref[...], preferred_element_type=f32)
        for hd in range(2):
            fslice = ft[:, hd * H:(hd + 1) * H]
            els[hd] = els[hd] + jnp.dot(fslice, a0_ref[0, hd, t][:, None],
                                        preferred_element_type=f32)
            ers[hd] = ers[hd] + jnp.dot(fslice, a0_ref[1, hd, t][:, None],
                                        preferred_element_type=f32)
    hca_ref[...] = jnp.stack([hc[:, :H // 2], hc[:, :H // 2]], axis=0)
    hcb_ref[...] = jnp.stack([hc[:, H // 2:], hc[:, H // 2:]], axis=0)
    elw_ref[...] = jnp.concatenate(
        els + ers + [jnp.zeros((BN, 4), f32)], axis=1)


def _tc2_body(agga_ref, aggb_ref, w0_ref, w1_ref, a1_ref,
              h1_ref, feata_ref, featb_ref, elw_ref):
    h1_parts = []
    for hd in range(2):
        a = jnp.concatenate([agga_ref[hd], aggb_ref[hd]], axis=1)
        v = jnp.dot(a, w0_ref[...][:, hd * H:(hd + 1) * H],
                    preferred_element_type=f32)
        h1_parts.append(_elu(v).reshape(BN, NT, H))
    h1 = jnp.concatenate(h1_parts, axis=2)
    h1_ref[...] = h1
    els = [jnp.zeros((BN, 1), f32) for _ in range(2)]
    ers = [jnp.zeros((BN, 1), f32) for _ in range(2)]
    fheads = [[], []]
    for t in range(NT):
        ft = jnp.dot(h1[:, t, :], w1_ref[...], preferred_element_type=f32)
        for hd in range(2):
            fslice = ft[:, hd * H:(hd + 1) * H]
            fheads[hd].append(fslice)
            els[hd] = els[hd] + jnp.dot(fslice, a1_ref[0, hd, t][:, None],
                                        preferred_element_type=f32)
            ers[hd] = ers[hd] + jnp.dot(fslice, a1_ref[1, hd, t][:, None],
                                        preferred_element_type=f32)
    f0 = jnp.concatenate(fheads[0], axis=1)
    f1 = jnp.concatenate(fheads[1], axis=1)
    hw = NT * H // 2
    feata_ref[...] = jnp.stack([f0[:, :hw], f1[:, :hw]], axis=0)
    featb_ref[...] = jnp.stack([f0[:, hw:], f1[:, hw:]], axis=0)
    elw_ref[...] = jnp.concatenate(
        els + ers + [jnp.zeros((BN, 4), f32)], axis=1)


def _tc3_body(agga_ref, aggb_ref, h1_ref, wres1_ref, w2_ref, a2_ref,
              h2_ref, feat_ref, elw_ref):
    h1 = h1_ref[...]
    h2_parts = []
    for hd in range(2):
        a = jnp.concatenate([agga_ref[hd], aggb_ref[hd]], axis=1)
        cols = []
        for t in range(NT):
            res = jnp.dot(h1[:, t, :],
                          wres1_ref[...][:, hd * H:(hd + 1) * H],
                          preferred_element_type=f32)
            cols.append(_elu(a[:, t * H:(t + 1) * H] + res)[:, None, :])
        h2_parts.append(jnp.concatenate(cols, axis=1))
    h2 = jnp.concatenate(h2_parts, axis=2)
    h2_ref[...] = h2
    el = jnp.zeros((BN, 1), f32)
    er = jnp.zeros((BN, 1), f32)
    fcols = []
    for t in range(NT):
        ft = jnp.dot(h2[:, t, :], w2_ref[...], preferred_element_type=f32)
        fcols.append(ft)
        el = el + jnp.dot(ft, a2_ref[0, 0, t][:, None],
                          preferred_element_type=f32)
        er = er + jnp.dot(ft, a2_ref[1, 0, t][:, None],
                          preferred_element_type=f32)
    fcat = jnp.concatenate(fcols + [jnp.zeros((BN, 32 - NT * NC), f32)],
                           axis=1)
    feat_ref[...] = jnp.stack([fcat, fcat], axis=0)
    elw_ref[...] = jnp.concatenate(
        [el, el, er, er, jnp.zeros((BN, 4), f32)], axis=1)


def _tc4_body(agg_ref, h2_ref, wres2_ref, out_ref):
    s = agg_ref[0][:, :NT * NC] + agg_ref[1][:, :NT * NC]
    h2 = h2_ref[...]
    acc = jnp.zeros((BN, NC), f32)
    for t in range(NT):
        res = jnp.dot(h2[:, t, :], wres2_ref[...], preferred_element_type=f32)
        acc = acc + s[:, t * NC:(t + 1) * NC] + res
    out_ref[...] = acc * (1.0 / NT)


def _full(shape):
    return pl.BlockSpec(shape, lambda i: tuple(0 for _ in shape))


def _tc1(xp, ntp2, W_fc, b_fc, W0, a0r):
    return pl.pallas_call(
        _tc1_body,
        grid=(GRID,),
        in_specs=[
            pl.BlockSpec((BN, D_IN), lambda i: (i, 0)),
            pl.BlockSpec((BN, 1), lambda i: (i, 0)),
            _full((NT, D_IN, H)),
            _full((NT, H)),
            _full((H, 2 * H)),
            _full((2, 2, NT, H)),
        ],
        out_specs=[
            pl.BlockSpec((2, BN, H // 2), lambda i: (0, i, 0)),
            pl.BlockSpec((2, BN, H // 2), lambda i: (0, i, 0)),
            pl.BlockSpec((BN, 8), lambda i: (i, 0)),
        ],
        out_shape=[
            jax.ShapeDtypeStruct((2, NPAD, H // 2), f32),
            jax.ShapeDtypeStruct((2, NPAD, H // 2), f32),
            jax.ShapeDtypeStruct((NPAD, 8), f32),
        ],
    )(xp, ntp2, W_fc, b_fc, W0, a0r)


def _tc2(agg0a, agg0b, W0, W1, a1r):
    hw = NT * H // 2
    return pl.pallas_call(
        _tc2_body,
        grid=(GRID,),
        in_specs=[
            pl.BlockSpec((2, NT * BN, H // 2), lambda i: (0, i, 0)),
            pl.BlockSpec((2, NT * BN, H // 2), lambda i: (0, i, 0)),
            _full((H, 2 * H)),
            _full((2 * H, 2 * H)),
            _full((2, 2, NT, H)),
        ],
        out_specs=[
            pl.BlockSpec((BN, NT, 2 * H), lambda i: (i, 0, 0)),
            pl.BlockSpec((2, BN, hw), lambda i: (0, i, 0)),
            pl.BlockSpec((2, BN, hw), lambda i: (0, i, 0)),
            pl.BlockSpec((BN, 8), lambda i: (i, 0)),
        ],
        out_shape=[
            jax.ShapeDtypeStruct((NPAD, NT, 2 * H), f32),
            jax.ShapeDtypeStruct((2, NPAD, hw), f32),
            jax.ShapeDtypeStruct((2, NPAD, hw), f32),
            jax.ShapeDtypeStruct((NPAD, 8), f32),
        ],
    )(agg0a, agg0b, W0, W1, a1r)


def _tc3(agg1a, agg1b, h1, Wres1, W2, a2r):
    hw = NT * H // 2
    return pl.pallas_call(
        _tc3_body,
        grid=(GRID,),
        in_specs=[
            pl.BlockSpec((2, BN, hw), lambda i: (0, i, 0)),
            pl.BlockSpec((2, BN, hw), lambda i: (0, i, 0)),
            pl.BlockSpec((BN, NT, 2 * H), lambda i: (i, 0, 0)),
            _full((2 * H, 2 * H)),
            _full((2 * H, NC)),
            _full((2, 1, NT, NC)),
        ],
        out_specs=[
            pl.BlockSpec((BN, NT, 2 * H), lambda i: (i, 0, 0)),
            pl.BlockSpec((2, BN, 32), lambda i: (0, i, 0)),
            pl.BlockSpec((BN, 8), lambda i: (i, 0)),
        ],
        out_shape=[
            jax.ShapeDtypeStruct((NPAD, NT, 2 * H), f32),
            jax.ShapeDtypeStruct((2, NPAD, 32), f32),
            jax.ShapeDtypeStruct((NPAD, 8), f32),
        ],
    )(agg1a, agg1b, h1, Wres1, W2, a2r)


def _tc4(agg2, h2, Wres2):
    return pl.pallas_call(
        _tc4_body,
        grid=(GRID,),
        in_specs=[
            pl.BlockSpec((2, BN, 32), lambda i: (0, i, 0)),
            pl.BlockSpec((BN, NT, 2 * H), lambda i: (i, 0, 0)),
            _full((2 * H, NC)),
        ],
        out_specs=pl.BlockSpec((BN, NC), lambda i: (i, 0)),
        out_shape=jax.ShapeDtypeStruct((NPAD, NC), f32),
    )(agg2, h2, Wres2)


def _stab(elw, eeT, nheads):
    elT = elw[:, 0:2].T
    erT = elw[:, 2:4].T
    m = (jnp.max(elT, axis=1, keepdims=True)
         + jnp.max(eeT[:, :NE], axis=1, keepdims=True))
    z = m + erT
    cT = jnp.where(z > 0, z, NEG * z)
    return elT, erT, cT


def kernel(x, node_type, edge_index, e_feat, W_fc, b_fc, W0, a0, ee0,
           W1, a1, ee1, Wres1, W2, a2, ee2, Wres2):
    src = edge_index[0]
    dst = edge_index[1]

    pad = EPAD - E
    src2d = jnp.concatenate([src, jnp.zeros((pad,), i32)]).reshape(-1, CH)
    dst2d = jnp.concatenate([dst, jnp.zeros((pad,), i32)]).reshape(-1, CH)
    et2d = jnp.concatenate(
        [e_feat, jnp.full((pad,), NE, i32)]).reshape(-1, CH)

    xp = jnp.concatenate([x, jnp.zeros((NPAD - N, D_IN), f32)], axis=0)
    ntp = jnp.concatenate([node_type, jnp.zeros((NPAD - N,), i32)])
    ntp2 = ntp[:, None]

    def ee_table(ee, heads):
        cols = [ee[:, hd if heads == 2 else 0][None, :] for hd in range(2)]
        t = jnp.concatenate(cols, axis=0)
        return jnp.concatenate([t, jnp.full((2, 8 - NE), -1e30, f32)], axis=1)

    a0r = a0.reshape(2, 2, NT, H)
    a1r = a1.reshape(2, 2, NT, H)
    a2r = a2.reshape(2, 1, NT, NC)

    hca, hcb, elw0 = _tc1(xp, ntp2, W_fc, b_fc, W0, a0r)
    ee0T = ee_table(ee0, 2)
    elT0, erT0, cT0 = _stab(elw0, ee0T, 2)
    alpha0, jidx0 = _sc_attn(True, src2d, dst2d, et2d, ntp,
                             elT0, erT0, cT0, ee0T)
    agg0a = _sc_msg(H // 2, NT * NPAD, src2d, jidx0, alpha0, hca)
    agg0b = _sc_msg(H // 2, NT * NPAD, src2d, jidx0, alpha0, hcb)

    h1, feat1a, feat1b, elw1 = _tc2(agg0a, agg0b, W0, W1, a1r)
    ee1T = ee_table(ee1, 2)
    elT1, erT1, cT1 = _stab(elw1, ee1T, 2)
    alpha1 = _sc_attn(False, src2d, dst2d, et2d, elT1, erT1, cT1, ee1T)
    agg1a = _sc_msg(NT * H // 2, NPAD, src2d, dst2d, alpha1, feat1a)
    agg1b = _sc_msg(NT * H // 2, NPAD, src2d, dst2d, alpha1, feat1b)

    h2, feat2, elw2 = _tc3(agg1a, agg1b, h1, Wres1, W2, a2r)
    ee2T = ee_table(ee2, 1)
    elT2, erT2, cT2 = _stab(elw2, ee2T, 1)
    agg2 = _sc_layer_post(32, RS // 2, RS // 2, src2d, dst2d, et2d,
                          elT2, erT2, cT2, ee2T, feat2)

    out = _tc4(agg2, h2, Wres2)
    return out[:N]
